# baseline (speedup 1.0000x reference)
import functools

import jax
import jax.numpy as jnp
import numpy as np
from jax import lax
from jax.experimental import pallas as pl
from jax.experimental.pallas import tpu as pltpu
from jax.experimental.pallas import tpu_sc as plsc

N = 10000
E = 320000
F_IN = 128
F_OUT = 128
T = 4
D = T * F_OUT

NC = 2
NS = 16
NW = NC * NS
EPW = E // NW
CHUNK = 40
NCHUNK = EPW // CHUNK
NPAD = 10240
ROWS_PER_SUB = NPAD // NS


def _matmul_y(x, kcat):
    def body(x_ref, k_ref, o_ref):
        o_ref[...] = jnp.dot(x_ref[...], k_ref[...],
                             preferred_element_type=jnp.float32
                             ).astype(jnp.bfloat16)

    return pl.pallas_call(
        body,
        grid=(10,),
        in_specs=[
            pl.BlockSpec((N // 10, F_IN), lambda i: (i, 0)),
            pl.BlockSpec((F_IN, D), lambda i: (0, 0)),
        ],
        out_specs=pl.BlockSpec((N // 10, D), lambda i: (i, 0)),
        out_shape=jax.ShapeDtypeStruct((N, D), jnp.bfloat16),
    )(x, kcat)


def _epilogue(partials, bias_row):
    def body(p_ref, b_ref, o_ref):
        o_ref[...] = jnp.maximum(p_ref[0] + p_ref[1] + b_ref[...], 0.0)

    return pl.pallas_call(
        body,
        grid=(10,),
        in_specs=[
            pl.BlockSpec((2, N // 10, F_OUT), lambda i: (0, i, 0)),
            pl.BlockSpec((1, F_OUT), lambda i: (0, 0)),
        ],
        out_specs=pl.BlockSpec((N // 10, F_OUT), lambda i: (i, 0)),
        out_shape=jax.ShapeDtypeStruct((N, F_OUT), jnp.float32),
    )(partials, bias_row)


def _make_sc_kernel():
    mesh = plsc.VectorSubcoreMesh(core_axis_name="c", subcore_axis_name="s",
                                  num_cores=NC, num_subcores=NS)

    @functools.partial(
        pl.kernel,
        out_type=jax.ShapeDtypeStruct((NC, NPAD, F_OUT), jnp.float32),
        mesh=mesh,
        compiler_params=pltpu.CompilerParams(needs_layout_passes=False),
        scratch_types=[
            pltpu.VMEM((CHUNK,), jnp.int32),
            pltpu.VMEM((CHUNK,), jnp.int32),
            pltpu.VMEM((CHUNK,), jnp.int32),
            pltpu.VMEM((CHUNK,), jnp.int32),
            pltpu.VMEM((CHUNK * T + 16,), jnp.float32),
            pltpu.VMEM((CHUNK * T + 16,), jnp.float32),
            pltpu.VMEM((CHUNK, D // 2), jnp.int32),
            pltpu.VMEM((CHUNK, D // 2), jnp.int32),
            pltpu.VMEM((CHUNK, F_OUT), jnp.float32),
            pltpu.VMEM_SHARED((NPAD, F_OUT), jnp.float32),
            pltpu.SemaphoreType.DMA,
            pltpu.SemaphoreType.DMA,
            pltpu.SemaphoreType.DMA,
            pltpu.SemaphoreType.DMA,
        ],
    )
    def sc_kernel(y_hbm, cols_hbm, rows_hbm, w_hbm, out_hbm,
                  cidx0_v, cidx1_v, ridx0_v, ridx1_v, w0_v, w1_v,
                  g0_v, g1_v, o_v, acc_sh,
                  isem0, isem1, gsem0, gsem1):
        cid = lax.axis_index("c")
        sid = lax.axis_index("s")
        cidx_b = (cidx0_v, cidx1_v)
        ridx_b = (ridx0_v, ridx1_v)
        w_b = (w0_v, w1_v)
        g_b = (g0_v, g1_v)
        isem = (isem0, isem1)
        gsem = (gsem0, gsem1)

        zero16 = jnp.zeros((16,), jnp.float32)

        def zinit(i, _):
            for j in range(F_OUT // 16):
                o_v[i, pl.ds(j * 16, 16)] = zero16
            return 0

        lax.fori_loop(0, CHUNK, zinit, 0)
        for zc in range(ROWS_PER_SUB // CHUNK):
            r0 = sid * ROWS_PER_SUB + zc * CHUNK
            pltpu.sync_copy(o_v, acc_sh.at[pl.ds(r0, CHUNK), :])
        plsc.subcore_barrier()

        wid = sid * NC + cid

        def idx_copies(i, b):
            base = wid * EPW + i * CHUNK
            return (
                pltpu.make_async_copy(cols_hbm.at[pl.ds(base, CHUNK)],
                                      cidx_b[b], isem[b]),
                pltpu.make_async_copy(rows_hbm.at[pl.ds(base, CHUNK)],
                                      ridx_b[b], isem[b]),
                pltpu.make_async_copy(
                    w_hbm.at[pl.ds(base * T, CHUNK * T)],
                    w_b[b].at[pl.ds(0, CHUNK * T)], isem[b]),
            )

        def issue_idx(i, b):
            for c in idx_copies(i, b):
                c.start()

        def wait_idx(i, b):
            for c in idx_copies(i, b):
                c.wait()

        def g_copy(b):
            return pltpu.make_async_copy(y_hbm.at[cidx_b[b]],
                                         g_b[b], gsem[b])

        def compute_scatter(b):
            g_v = g_b[b]

            himask = jnp.full((16,), -65536, jnp.int32)

            @plsc.parallel_loop(0, CHUNK, unroll=8)
            def edge_body(e):
                wv = w_b[b][pl.ds(e * T, 16)]
                ws = (wv[0], wv[1], wv[2], wv[3])
                for k in range(F_OUT // 32):
                    acc_lo = jnp.zeros((16,), jnp.float32)
                    acc_hi = jnp.zeros((16,), jnp.float32)
                    for t in range(T):
                        v = g_v[e, pl.ds(t * (F_OUT // 2) + k * 16, 16)]
                        lo = plsc.bitcast(lax.shift_left(v, 16), jnp.float32)
                        hi = plsc.bitcast(v & himask, jnp.float32)
                        acc_lo = acc_lo + ws[t] * lo
                        acc_hi = acc_hi + ws[t] * hi
                    o_v[e, pl.ds(k * 32, 16)] = acc_lo
                    o_v[e, pl.ds(k * 32 + 16, 16)] = acc_hi
            pltpu.sync_copy(o_v, acc_sh.at[ridx_b[b]], add=True)

        issue_idx(0, 0)
        wait_idx(0, 0)
        g_copy(0).start()
        issue_idx(1, 1)

        def pair_body(ip, _):
            for b in (0, 1):
                i = ip * 2 + b
                g_copy(b).wait()
                wait_idx(i + 1, 1 - b)
                g_copy(1 - b).start()
                compute_scatter(b)
                issue_idx(i + 2, b)
            return 0

        lax.fori_loop(0, (NCHUNK - 2) // 2, pair_body, 0)
        g_copy(0).wait()
        wait_idx(NCHUNK - 1, 1)
        g_copy(1).start()
        compute_scatter(0)
        g_copy(1).wait()
        compute_scatter(1)

        plsc.subcore_barrier()
        r0 = sid * ROWS_PER_SUB
        pltpu.sync_copy(acc_sh.at[pl.ds(r0, ROWS_PER_SUB), :],
                        out_hbm.at[cid, pl.ds(r0, ROWS_PER_SUB), :])

    return sc_kernel


def kernel(node_features, edge_features, indices, out_size, kernel, bias):
    kcat = jnp.transpose(kernel, (1, 0, 2)).reshape(F_IN, D)
    blk = np.empty((32,), np.int64)
    blk[0::2] = np.arange(16)
    blk[1::2] = np.arange(16) + 16
    perm = (np.arange(D) // 32) * 32 + blk[np.arange(D) % 32]
    kcat = kcat[:, perm]
    cols = indices[:, 1]
    rows = indices[:, 0]
    w_em = edge_features.T.reshape(-1)
    bias_adj = bias + (jnp.asarray(out_size, jnp.float32) - float(N))

    y = _matmul_y(node_features, kcat)
    y_i32 = lax.bitcast_convert_type(y.reshape(N, D // 2, 2), jnp.int32)
    partials = _make_sc_kernel()(y_i32, cols, rows, w_em)
    return _epilogue(partials, bias_adj.reshape(1, F_OUT))


# speedup vs baseline: 1.2838x; 1.2838x over previous
import functools

import jax
import jax.numpy as jnp
import numpy as np
from jax import lax
from jax.experimental import pallas as pl
from jax.experimental.pallas import tpu as pltpu
from jax.experimental.pallas import tpu_sc as plsc

N = 10000
E = 320000
F_IN = 128
F_OUT = 128
T = 4
D = T * F_OUT

NC = 2
NS = 16
NW = NC * NS
EPW = E // NW
CHUNK = 40
NCHUNK = EPW // CHUNK
NPAD = 10240
ROWS_PER_SUB = NPAD // NS


def _matmul_y(x, kcat2):
    def body(x_ref, k_ref, o_ref):
        r = jnp.dot(x_ref[...], k_ref[...],
                    preferred_element_type=jnp.float32)
        ra = r[:, :D // 2].astype(jnp.bfloat16).astype(jnp.float32)
        rb = r[:, D // 2:].astype(jnp.bfloat16).astype(jnp.float32)
        abits = lax.shift_right_logical(
            lax.bitcast_convert_type(ra, jnp.int32), 16)
        bbits = lax.bitcast_convert_type(rb, jnp.int32) & jnp.int32(-65536)
        o_ref[...] = abits | bbits

    return pl.pallas_call(
        body,
        grid=(10,),
        in_specs=[
            pl.BlockSpec((N // 10, F_IN), lambda i: (i, 0)),
            pl.BlockSpec((F_IN, D), lambda i: (0, 0)),
        ],
        out_specs=pl.BlockSpec((N // 10, D // 2), lambda i: (i, 0)),
        out_shape=jax.ShapeDtypeStruct((N, D // 2), jnp.int32),
    )(x, kcat2)


def _epilogue(partials, bias_row):
    def body(p_ref, b_ref, o_ref):
        o_ref[...] = jnp.maximum(p_ref[0] + p_ref[1] + b_ref[...], 0.0)

    return pl.pallas_call(
        body,
        grid=(10,),
        in_specs=[
            pl.BlockSpec((2, N // 10, F_OUT), lambda i: (0, i, 0)),
            pl.BlockSpec((1, F_OUT), lambda i: (0, 0)),
        ],
        out_specs=pl.BlockSpec((N // 10, F_OUT), lambda i: (i, 0)),
        out_shape=jax.ShapeDtypeStruct((N, F_OUT), jnp.float32),
    )(partials, bias_row)


def _make_sc_kernel():
    mesh = plsc.VectorSubcoreMesh(core_axis_name="c", subcore_axis_name="s",
                                  num_cores=NC, num_subcores=NS)

    @functools.partial(
        pl.kernel,
        out_type=jax.ShapeDtypeStruct((NC, NPAD, F_OUT), jnp.float32),
        mesh=mesh,
        compiler_params=pltpu.CompilerParams(needs_layout_passes=False),
        scratch_types=[
            pltpu.VMEM((CHUNK,), jnp.int32),
            pltpu.VMEM((CHUNK,), jnp.int32),
            pltpu.VMEM((CHUNK,), jnp.int32),
            pltpu.VMEM((CHUNK,), jnp.int32),
            pltpu.VMEM((CHUNK * T + 16,), jnp.float32),
            pltpu.VMEM((CHUNK * T + 16,), jnp.float32),
            pltpu.VMEM((CHUNK, D // 2), jnp.int32),
            pltpu.VMEM((CHUNK, D // 2), jnp.int32),
            pltpu.VMEM((CHUNK, F_OUT), jnp.float32),
            pltpu.VMEM_SHARED((NPAD, F_OUT), jnp.float32),
            pltpu.SemaphoreType.DMA,
            pltpu.SemaphoreType.DMA,
            pltpu.SemaphoreType.DMA,
            pltpu.SemaphoreType.DMA,
        ],
    )
    def sc_kernel(y_hbm, cols_hbm, rows_hbm, w_hbm, out_hbm,
                  cidx0_v, cidx1_v, ridx0_v, ridx1_v, w0_v, w1_v,
                  g0_v, g1_v, o_v, acc_sh,
                  isem0, isem1, gsem0, gsem1):
        cid = lax.axis_index("c")
        sid = lax.axis_index("s")
        cidx_b = (cidx0_v, cidx1_v)
        ridx_b = (ridx0_v, ridx1_v)
        w_b = (w0_v, w1_v)
        g_b = (g0_v, g1_v)
        isem = (isem0, isem1)
        gsem = (gsem0, gsem1)

        zero16 = jnp.zeros((16,), jnp.float32)

        def zinit(i, _):
            for j in range(F_OUT // 16):
                o_v[i, pl.ds(j * 16, 16)] = zero16
            return 0

        lax.fori_loop(0, CHUNK, zinit, 0)
        for zc in range(ROWS_PER_SUB // CHUNK):
            r0 = sid * ROWS_PER_SUB + zc * CHUNK
            pltpu.sync_copy(o_v, acc_sh.at[pl.ds(r0, CHUNK), :])
        plsc.subcore_barrier()

        wid = sid * NC + cid

        def idx_copies(i, b):
            base = wid * EPW + i * CHUNK
            return (
                pltpu.make_async_copy(cols_hbm.at[pl.ds(base, CHUNK)],
                                      cidx_b[b], isem[b]),
                pltpu.make_async_copy(rows_hbm.at[pl.ds(base, CHUNK)],
                                      ridx_b[b], isem[b]),
                pltpu.make_async_copy(
                    w_hbm.at[pl.ds(base * T, CHUNK * T)],
                    w_b[b].at[pl.ds(0, CHUNK * T)], isem[b]),
            )

        def issue_idx(i, b):
            for c in idx_copies(i, b):
                c.start()

        def wait_idx(i, b):
            for c in idx_copies(i, b):
                c.wait()

        def g_copy(b):
            return pltpu.make_async_copy(y_hbm.at[cidx_b[b]],
                                         g_b[b], gsem[b])

        def compute_scatter(b):
            g_v = g_b[b]

            himask = jnp.full((16,), -65536, jnp.int32)

            @plsc.parallel_loop(0, CHUNK, unroll=8)
            def edge_body(e):
                wv = w_b[b][pl.ds(e * T, 16)]
                ws = (wv[0], wv[1], wv[2], wv[3])
                for k in range(F_OUT // 32):
                    acc_lo = jnp.zeros((16,), jnp.float32)
                    acc_hi = jnp.zeros((16,), jnp.float32)
                    for t in range(T):
                        v = g_v[e, pl.ds(t * (F_OUT // 2) + k * 16, 16)]
                        lo = plsc.bitcast(lax.shift_left(v, 16), jnp.float32)
                        hi = plsc.bitcast(v & himask, jnp.float32)
                        acc_lo = acc_lo + ws[t] * lo
                        acc_hi = acc_hi + ws[t] * hi
                    o_v[e, pl.ds(k * 32, 16)] = acc_lo
                    o_v[e, pl.ds(k * 32 + 16, 16)] = acc_hi
            pltpu.sync_copy(o_v, acc_sh.at[ridx_b[b]], add=True)

        issue_idx(0, 0)
        wait_idx(0, 0)
        g_copy(0).start()
        issue_idx(1, 1)

        def pair_body(ip, _):
            for b in (0, 1):
                i = ip * 2 + b
                g_copy(b).wait()
                wait_idx(i + 1, 1 - b)
                g_copy(1 - b).start()
                compute_scatter(b)
                issue_idx(i + 2, b)
            return 0

        lax.fori_loop(0, (NCHUNK - 2) // 2, pair_body, 0)
        g_copy(0).wait()
        wait_idx(NCHUNK - 1, 1)
        g_copy(1).start()
        compute_scatter(0)
        g_copy(1).wait()
        compute_scatter(1)

        plsc.subcore_barrier()
        r0 = sid * ROWS_PER_SUB
        pltpu.sync_copy(acc_sh.at[pl.ds(r0, ROWS_PER_SUB), :],
                        out_hbm.at[cid, pl.ds(r0, ROWS_PER_SUB), :])

    return sc_kernel


def kernel(node_features, edge_features, indices, out_size, kernel, bias):
    kcat = jnp.transpose(kernel, (1, 0, 2)).reshape(F_IN, D)
    m = np.arange(D // 2)
    perm_a = (m // 64) * 128 + 32 * ((m % 64) // 16) + (m % 16)
    kcat2 = jnp.concatenate([kcat[:, perm_a], kcat[:, perm_a + 16]], axis=1)
    cols = indices[:, 1]
    rows = indices[:, 0]
    w_em = edge_features.T.reshape(-1)
    bias_adj = bias + (jnp.asarray(out_size, jnp.float32) - float(N))

    y_i32 = _matmul_y(node_features, kcat2)
    partials = _make_sc_kernel()(y_i32, cols, rows, w_em)
    return _epilogue(partials, bias_adj.reshape(1, F_OUT))
